# baseline (speedup 1.0000x reference)
import functools

import jax
import jax.numpy as jnp
from jax import lax
from jax.experimental import pallas as pl
from jax.experimental.pallas import tpu as pltpu
from jax.experimental.pallas import tpu_sc as plsc

N = 10000
E = 320000
D = 128
H = 128
C = 40
CP = 128

NCORES = 2
NSUB = 16
NW = NCORES * NSUB
CHUNK = 128
NBUF = 2
TOTC = 160
CPW0 = 80
CPW1 = TOTC - CPW0
NCH = NSUB * TOTC
EP = NCH * CHUNK
NACC = 10240
RPT = NACC // NSUB
TRASH = N

_mesh = plsc.VectorSubcoreMesh(core_axis_name="c", subcore_axis_name="s")


def _make_scatter(F, gather, nc0=CPW0, nc1=CPW1):
    @functools.partial(
        pl.kernel,
        out_type=jax.ShapeDtypeStruct((NCORES, NACC, F), jnp.float32),
        mesh=_mesh,
        scratch_types=[
            pltpu.VMEM((NBUF, 2, CHUNK), jnp.int32),
            pltpu.VMEM((NBUF, CHUNK, F), jnp.float32),
            pltpu.VMEM_SHARED((NACC, F), jnp.float32),
            pltpu.SemaphoreType.DMA,
            pltpu.SemaphoreType.DMA,
            pltpu.SemaphoreType.DMA,
            pltpu.SemaphoreType.DMA,
        ],
    )
    def scat(table_hbm, sd_hbm, zeros_hbm, out_hbm,
             idx, rows, acc, semg0, semg1, semi0, semi1):
        c = lax.axis_index("c")
        s = lax.axis_index("s")
        r0 = s * RPT
        base = s * TOTC + c * nc0
        semg = (semg0, semg1)
        semi = (semi0, semi1)

        def fetch_idx(j, b):
            pltpu.async_copy(sd_hbm.at[base + j], idx.at[b], semi[b])

        def wait_idx(b):
            pltpu.make_async_copy(sd_hbm.at[0], idx.at[b], semi[b]).wait()

        def fetch_rows(b):
            pltpu.async_copy(table_hbm.at[idx.at[b, 0]], rows.at[b], semg[b])

        def wait_rows(b):
            pltpu.make_async_copy(table_hbm.at[idx.at[0, 0]], rows.at[b],
                                  semg[b]).wait()

        def scatter_add(rb, ib):
            pltpu.sync_copy(rows.at[rb], acc.at[idx.at[ib, 1]], add=True)

        def pipelined(nch):
            pltpu.sync_copy(sd_hbm.at[base], idx.at[0])
            fetch_rows(0)
            fetch_idx(1, 1)

            def body(g, carry):
                for b in range(2):
                    j = g * 2 + b
                    bn = 1 - b
                    wait_idx(bn)
                    fetch_rows(bn)
                    wait_rows(b)
                    scatter_add(b, b)
                    fetch_idx(jnp.minimum(j + 2, nch - 1), b)
                return carry

            lax.fori_loop(0, nch // 2, body, 0)
            wait_rows(0)
            wait_idx(1)

        def serial(nch):
            fetch_idx(0, 0)
            fetch_idx(1, 1)

            def body(g, carry):
                for b in range(2):
                    j = g * 2 + b
                    wait_idx(b)
                    if gather:
                        pltpu.async_copy(table_hbm.at[idx.at[b, 0]],
                                         rows.at[b], semg[b]).wait()
                    scatter_add(b if gather else 0, b)
                    fetch_idx(jnp.minimum(j + 2, nch - 1), b)
                return carry

            lax.fori_loop(0, nch // 2, body, 0)
            wait_idx(0)
            wait_idx(1)

        pltpu.sync_copy(zeros_hbm.at[pl.ds(r0, RPT)], acc.at[pl.ds(r0, RPT)])
        if not gather:
            pltpu.sync_copy(table_hbm.at[pl.ds(0, CHUNK)], rows.at[0])
        plsc.subcore_barrier()

        if gather:
            @pl.when(c == 0)
            def _():
                serial(nc0)

            @pl.when(c == 1)
            def _():
                serial(nc1)
        else:
            @pl.when(c == 0)
            def _():
                serial(nc0)

            @pl.when(c == 1)
            def _():
                serial(nc1)
        plsc.subcore_barrier()
        pltpu.sync_copy(acc.at[pl.ds(r0, RPT)],
                        out_hbm.at[c, pl.ds(r0, RPT)])

    return scat


_scat = _make_scatter(H, gather=True)
_deg_kernel = _make_scatter(H, gather=False, nc0=TOTC // 2, nc1=TOTC // 2)

_BN = 1000


def _dis_block(deg_ref):
    d = deg_ref[0, :, 0:1] + deg_ref[1, :, 0:1] + 1.0
    return lax.rsqrt(d)


def _tc1_body(x_ref, w_ref, deg_ref, y_ref):
    dis = _dis_block(deg_ref)
    y_ref[...] = jnp.dot(x_ref[...], w_ref[...],
                         preferred_element_type=jnp.float32) * dis


def _tc2_body(z_ref, y1_ref, deg_ref, b1_ref, w2_ref, y2_ref):
    dis = _dis_block(deg_ref)
    h = (z_ref[0] + z_ref[1] + y1_ref[...]) * dis + b1_ref[...]
    h = jnp.maximum(h, 0.0)
    y2_ref[...] = jnp.dot(h, w2_ref[...],
                          preferred_element_type=jnp.float32) * dis


def _tc3_body(z_ref, y2_ref, deg_ref, b2_ref, out_ref):
    dis = _dis_block(deg_ref)
    v = (z_ref[0] + z_ref[1] + y2_ref[...]) * dis + b2_ref[...]
    col = lax.broadcasted_iota(jnp.int32, v.shape, 1)
    valid = col < C
    m = jnp.max(jnp.where(valid, v, -1e30), axis=1, keepdims=True)
    e = jnp.where(valid, jnp.exp(v - m), 0.0)
    lse = jnp.log(jnp.sum(e, axis=1, keepdims=True)) + m
    out_ref[...] = v - lse


def _deg_spec():
    return pl.BlockSpec((NCORES, _BN, H), lambda i: (0, i, 0))


def kernel(x, edge_index, W1, b1, W2, b2):
    src = edge_index[0].astype(jnp.int32)
    dst = edge_index[1].astype(jnp.int32)
    pad = EP - E
    srcw = jnp.concatenate([src, jnp.zeros((pad,), jnp.int32)]
                           ).reshape(NCH, CHUNK)
    dstw = jnp.concatenate([dst, jnp.full((pad,), TRASH, jnp.int32)]
                           ).reshape(NCH, CHUNK)
    sd = jnp.stack([srcw, dstw], axis=1)

    zeros128 = jnp.zeros((NACC, H), jnp.float32)
    ones128 = jnp.ones((CHUNK, H), jnp.float32)
    degp = _deg_kernel(ones128, sd, zeros128)

    y1 = pl.pallas_call(
        _tc1_body,
        grid=(N // _BN,),
        in_specs=[pl.BlockSpec((_BN, D), lambda i: (i, 0)),
                  pl.BlockSpec((D, H), lambda i: (0, 0)),
                  _deg_spec()],
        out_specs=pl.BlockSpec((_BN, H), lambda i: (i, 0)),
        out_shape=jax.ShapeDtypeStruct((N, H), jnp.float32),
    )(x, W1, degp)

    z1p = _scat(y1, sd, zeros128)

    W2p = jnp.pad(W2, ((0, 0), (0, CP - C)))
    y2 = pl.pallas_call(
        _tc2_body,
        grid=(N // _BN,),
        in_specs=[pl.BlockSpec((NCORES, _BN, H), lambda i: (0, i, 0)),
                  pl.BlockSpec((_BN, H), lambda i: (i, 0)),
                  _deg_spec(),
                  pl.BlockSpec((1, H), lambda i: (0, 0)),
                  pl.BlockSpec((H, CP), lambda i: (0, 0))],
        out_specs=pl.BlockSpec((_BN, CP), lambda i: (i, 0)),
        out_shape=jax.ShapeDtypeStruct((N, CP), jnp.float32),
    )(z1p, y1, degp, b1.reshape(1, H), W2p)

    z2p = _scat(y2, sd, zeros128)

    b2p = jnp.pad(b2, (0, CP - C)).reshape(1, CP)
    out = pl.pallas_call(
        _tc3_body,
        grid=(N // _BN,),
        in_specs=[pl.BlockSpec((NCORES, _BN, CP), lambda i: (0, i, 0)),
                  pl.BlockSpec((_BN, CP), lambda i: (i, 0)),
                  _deg_spec(),
                  pl.BlockSpec((1, CP), lambda i: (0, 0))],
        out_specs=pl.BlockSpec((_BN, CP), lambda i: (i, 0)),
        out_shape=jax.ShapeDtypeStruct((N, CP), jnp.float32),
    )(z2p, y2, degp, b2p)

    return out[:, :C]


# speedup vs baseline: 1.4675x; 1.4675x over previous
import functools

import jax
import jax.numpy as jnp
from jax import lax
from jax.experimental import pallas as pl
from jax.experimental.pallas import tpu as pltpu
from jax.experimental.pallas import tpu_sc as plsc

N = 10000
E = 320000
D = 128
H = 128
C = 40
CP = 128

NCORES = 2
NSUB = 16
NW = NCORES * NSUB
CHUNK = 128
NBUF = 2
TOTC = 160
CPW = TOTC // NCORES
HC = CPW // 2
NCH = NSUB * TOTC
EP = NCH * CHUNK
NACC = 10112
RPT = NACC // NSUB
TRASH = N

_mesh = plsc.VectorSubcoreMesh(core_axis_name="c", subcore_axis_name="s")


def _make_scatter(F, gather):
    @functools.partial(
        pl.kernel,
        out_type=jax.ShapeDtypeStruct((NCORES, NACC, F), jnp.float32),
        mesh=_mesh,
        scratch_types=[
            pltpu.VMEM((HC, 2, CHUNK), jnp.int32),
            pltpu.VMEM((NBUF, CHUNK, F), jnp.float32),
            pltpu.VMEM_SHARED((NACC, F), jnp.float32),
            pltpu.SemaphoreType.DMA,
            pltpu.SemaphoreType.DMA,
        ],
    )
    def scat(table_hbm, sd_hbm, zeros_hbm, out_hbm,
             idx, rows, acc, semg0, semg1):
        c = lax.axis_index("c")
        s = lax.axis_index("s")
        r0 = s * RPT
        base = s * TOTC + c * CPW
        semg = (semg0, semg1)

        def fetch_rows(j, b):
            pltpu.async_copy(table_hbm.at[idx.at[j, 0]], rows.at[b], semg[b])

        def wait_rows(b):
            pltpu.make_async_copy(table_hbm.at[idx.at[0, 0]], rows.at[b],
                                  semg[b]).wait()

        pltpu.sync_copy(zeros_hbm.at[pl.ds(r0, RPT)], acc.at[pl.ds(r0, RPT)])
        if not gather:
            pltpu.sync_copy(table_hbm.at[pl.ds(0, CHUNK)], rows.at[0])
        plsc.subcore_barrier()

        for h in range(CPW // HC):
            pltpu.sync_copy(sd_hbm.at[pl.ds(base + h * HC, HC)], idx)
            if gather:
                fetch_rows(0, 0)
                fetch_rows(1, 1)

                def body(g, carry):
                    for b in range(2):
                        j = g * 2 + b
                        wait_rows(b)
                        pltpu.sync_copy(rows.at[b], acc.at[idx.at[j, 1]],
                                        add=True)
                        fetch_rows(jnp.minimum(j + 2, HC - 1), b)
                    return carry

                lax.fori_loop(0, HC // 2, body, 0)
                wait_rows(0)
                wait_rows(1)
            else:
                def dbody(j, carry):
                    pltpu.sync_copy(rows.at[0], acc.at[idx.at[j, 1]],
                                    add=True)
                    return carry

                lax.fori_loop(0, HC, dbody, 0)
        plsc.subcore_barrier()
        pltpu.sync_copy(acc.at[pl.ds(r0, RPT)],
                        out_hbm.at[c, pl.ds(r0, RPT)])

    return scat


_scat = _make_scatter(H, gather=True)
_deg_kernel = _make_scatter(H, gather=False)

_BN = 1000


def _dis_block(deg_ref):
    d = deg_ref[0, :, 0:1] + deg_ref[1, :, 0:1] + 1.0
    return lax.rsqrt(d)


def _tc1_body(x_ref, w_ref, deg_ref, y_ref):
    dis = _dis_block(deg_ref)
    y_ref[...] = jnp.dot(x_ref[...], w_ref[...],
                         preferred_element_type=jnp.float32) * dis


def _tc2_body(z_ref, y1_ref, deg_ref, b1_ref, w2_ref, y2_ref):
    dis = _dis_block(deg_ref)
    h = (z_ref[0] + z_ref[1] + y1_ref[...]) * dis + b1_ref[...]
    h = jnp.maximum(h, 0.0)
    y2_ref[...] = jnp.dot(h, w2_ref[...],
                          preferred_element_type=jnp.float32) * dis


def _tc3_body(z_ref, y2_ref, deg_ref, b2_ref, out_ref):
    dis = _dis_block(deg_ref)
    v = (z_ref[0] + z_ref[1] + y2_ref[...]) * dis + b2_ref[...]
    col = lax.broadcasted_iota(jnp.int32, v.shape, 1)
    valid = col < C
    m = jnp.max(jnp.where(valid, v, -1e30), axis=1, keepdims=True)
    e = jnp.where(valid, jnp.exp(v - m), 0.0)
    lse = jnp.log(jnp.sum(e, axis=1, keepdims=True)) + m
    out_ref[...] = v - lse


def _deg_spec():
    return pl.BlockSpec((NCORES, _BN, H), lambda i: (0, i, 0))


def kernel(x, edge_index, W1, b1, W2, b2):
    src = edge_index[0].astype(jnp.int32)
    dst = edge_index[1].astype(jnp.int32)
    pad = EP - E
    srcw = jnp.concatenate([src, jnp.zeros((pad,), jnp.int32)]
                           ).reshape(NCH, CHUNK)
    dstw = jnp.concatenate([dst, jnp.full((pad,), TRASH, jnp.int32)]
                           ).reshape(NCH, CHUNK)
    sd = jnp.stack([srcw, dstw], axis=1)

    zeros128 = jnp.zeros((NACC, H), jnp.float32)
    ones128 = jnp.ones((CHUNK, H), jnp.float32)
    degp = _deg_kernel(ones128, sd, zeros128)

    y1 = pl.pallas_call(
        _tc1_body,
        grid=(N // _BN,),
        in_specs=[pl.BlockSpec((_BN, D), lambda i: (i, 0)),
                  pl.BlockSpec((D, H), lambda i: (0, 0)),
                  _deg_spec()],
        out_specs=pl.BlockSpec((_BN, H), lambda i: (i, 0)),
        out_shape=jax.ShapeDtypeStruct((N, H), jnp.float32),
    )(x, W1, degp)

    z1p = _scat(y1, sd, zeros128)

    W2p = jnp.pad(W2, ((0, 0), (0, CP - C)))
    y2 = pl.pallas_call(
        _tc2_body,
        grid=(N // _BN,),
        in_specs=[pl.BlockSpec((NCORES, _BN, H), lambda i: (0, i, 0)),
                  pl.BlockSpec((_BN, H), lambda i: (i, 0)),
                  _deg_spec(),
                  pl.BlockSpec((1, H), lambda i: (0, 0)),
                  pl.BlockSpec((H, CP), lambda i: (0, 0))],
        out_specs=pl.BlockSpec((_BN, CP), lambda i: (i, 0)),
        out_shape=jax.ShapeDtypeStruct((N, CP), jnp.float32),
    )(z1p, y1, degp, b1.reshape(1, H), W2p)

    z2p = _scat(y2, sd, zeros128)

    b2p = jnp.pad(b2, (0, CP - C)).reshape(1, CP)
    out = pl.pallas_call(
        _tc3_body,
        grid=(N // _BN,),
        in_specs=[pl.BlockSpec((NCORES, _BN, CP), lambda i: (0, i, 0)),
                  pl.BlockSpec((_BN, CP), lambda i: (i, 0)),
                  _deg_spec(),
                  pl.BlockSpec((1, CP), lambda i: (0, 0))],
        out_specs=pl.BlockSpec((_BN, CP), lambda i: (i, 0)),
        out_shape=jax.ShapeDtypeStruct((N, CP), jnp.float32),
    )(z2p, y2, degp, b2p)

    return out[:, :C]


# speedup vs baseline: 1.5213x; 1.0366x over previous
import functools

import jax
import jax.numpy as jnp
from jax import lax
from jax.experimental import pallas as pl
from jax.experimental.pallas import tpu as pltpu
from jax.experimental.pallas import tpu_sc as plsc

N = 10000
E = 320000
D = 128
H = 128
C = 40
CP = 128

NCORES = 2
NSUB = 16
NW = NCORES * NSUB
CHUNK = 128
NBUF = 2
TOTC = 160
CPW = TOTC // NCORES
HC = CPW // 2
NCH = NSUB * TOTC
EP = NCH * CHUNK
NACC = 10112
RPT = NACC // NSUB
TRASH = N

_mesh = plsc.VectorSubcoreMesh(core_axis_name="c", subcore_axis_name="s")


def _make_scatter(F, gather, n0=CPW, n1=CPW):
    @functools.partial(
        pl.kernel,
        out_type=jax.ShapeDtypeStruct((NCORES, NACC, F), jnp.float32),
        mesh=_mesh,
        scratch_types=[
            pltpu.VMEM((HC, 2, CHUNK), jnp.int32),
            pltpu.VMEM((NBUF, CHUNK, F), jnp.float32),
            pltpu.VMEM_SHARED((NACC, F), jnp.float32),
            pltpu.SemaphoreType.DMA,
            pltpu.SemaphoreType.DMA,
        ],
    )
    def scat(table_hbm, sd_hbm, zeros_hbm, out_hbm,
             idx, rows, acc, semg0, semg1):
        c = lax.axis_index("c")
        s = lax.axis_index("s")
        r0 = s * RPT
        base = s * TOTC + c * n0
        nch = jnp.where(c == 0, n0, n1)
        semg = (semg0, semg1)

        def fetch_rows(j, b):
            pltpu.async_copy(table_hbm.at[idx.at[j, 0]], rows.at[b], semg[b])

        def wait_rows(b):
            pltpu.make_async_copy(table_hbm.at[idx.at[0, 0]], rows.at[b],
                                  semg[b]).wait()

        pltpu.sync_copy(zeros_hbm.at[pl.ds(r0, RPT)], acc.at[pl.ds(r0, RPT)])
        if not gather:
            pltpu.sync_copy(table_hbm.at[pl.ds(0, CHUNK)], rows.at[0])
        plsc.subcore_barrier()

        def run_half(h):
            pltpu.sync_copy(sd_hbm.at[pl.ds(base + h * HC, HC)], idx)
            if gather:
                fetch_rows(0, 0)
                fetch_rows(1, 1)

                def body(g, carry):
                    for b in range(2):
                        j = g * 2 + b
                        wait_rows(b)
                        pltpu.sync_copy(rows.at[b], acc.at[idx.at[j, 1]],
                                        add=True)
                        fetch_rows(jnp.minimum(j + 2, HC - 1), b)
                    return carry

                lax.fori_loop(0, HC // 2, body, 0)
                wait_rows(0)
                wait_rows(1)
            else:
                def dbody(j, carry):
                    pltpu.sync_copy(rows.at[0], acc.at[idx.at[j, 1]],
                                    add=True)
                    return carry

                lax.fori_loop(0, HC, dbody, 0)

        for h in range(max(n0, n1) // HC):
            if (h + 1) * HC <= min(n0, n1):
                run_half(h)
            else:
                @pl.when(h * HC < nch)
                def _():
                    run_half(h)
        plsc.subcore_barrier()
        pltpu.sync_copy(acc.at[pl.ds(r0, RPT)],
                        out_hbm.at[c, pl.ds(r0, RPT)])

    return scat


_scat = _make_scatter(H, gather=True, n0=120, n1=40)
_deg_kernel = _make_scatter(H, gather=False)

_BN = 1000


def _dis_block(deg_ref):
    d = deg_ref[0, :, 0:1] + deg_ref[1, :, 0:1] + 1.0
    return lax.rsqrt(d)


def _tc1_body(x_ref, w_ref, deg_ref, y_ref):
    dis = _dis_block(deg_ref)
    y_ref[...] = jnp.dot(x_ref[...], w_ref[...],
                         preferred_element_type=jnp.float32) * dis


def _tc2_body(z_ref, y1_ref, deg_ref, b1_ref, w2_ref, y2_ref):
    dis = _dis_block(deg_ref)
    h = (z_ref[0] + z_ref[1] + y1_ref[...]) * dis + b1_ref[...]
    h = jnp.maximum(h, 0.0)
    y2_ref[...] = jnp.dot(h, w2_ref[...],
                          preferred_element_type=jnp.float32) * dis


def _tc3_body(z_ref, y2_ref, deg_ref, b2_ref, out_ref):
    dis = _dis_block(deg_ref)
    v = (z_ref[0] + z_ref[1] + y2_ref[...]) * dis + b2_ref[...]
    col = lax.broadcasted_iota(jnp.int32, v.shape, 1)
    valid = col < C
    m = jnp.max(jnp.where(valid, v, -1e30), axis=1, keepdims=True)
    e = jnp.where(valid, jnp.exp(v - m), 0.0)
    lse = jnp.log(jnp.sum(e, axis=1, keepdims=True)) + m
    out_ref[...] = v - lse


def _deg_spec():
    return pl.BlockSpec((NCORES, _BN, H), lambda i: (0, i, 0))


def kernel(x, edge_index, W1, b1, W2, b2):
    src = edge_index[0].astype(jnp.int32)
    dst = edge_index[1].astype(jnp.int32)
    pad = EP - E
    srcw = jnp.concatenate([src, jnp.zeros((pad,), jnp.int32)]
                           ).reshape(NCH, CHUNK)
    dstw = jnp.concatenate([dst, jnp.full((pad,), TRASH, jnp.int32)]
                           ).reshape(NCH, CHUNK)
    sd = jnp.stack([srcw, dstw], axis=1)

    zeros128 = jnp.zeros((NACC, H), jnp.float32)
    ones128 = jnp.ones((CHUNK, H), jnp.float32)
    degp = _deg_kernel(ones128, sd, zeros128)

    y1 = pl.pallas_call(
        _tc1_body,
        grid=(N // _BN,),
        in_specs=[pl.BlockSpec((_BN, D), lambda i: (i, 0)),
                  pl.BlockSpec((D, H), lambda i: (0, 0)),
                  _deg_spec()],
        out_specs=pl.BlockSpec((_BN, H), lambda i: (i, 0)),
        out_shape=jax.ShapeDtypeStruct((N, H), jnp.float32),
    )(x, W1, degp)

    z1p = _scat(y1, sd, zeros128)

    W2p = jnp.pad(W2, ((0, 0), (0, CP - C)))
    y2 = pl.pallas_call(
        _tc2_body,
        grid=(N // _BN,),
        in_specs=[pl.BlockSpec((NCORES, _BN, H), lambda i: (0, i, 0)),
                  pl.BlockSpec((_BN, H), lambda i: (i, 0)),
                  _deg_spec(),
                  pl.BlockSpec((1, H), lambda i: (0, 0)),
                  pl.BlockSpec((H, CP), lambda i: (0, 0))],
        out_specs=pl.BlockSpec((_BN, CP), lambda i: (i, 0)),
        out_shape=jax.ShapeDtypeStruct((N, CP), jnp.float32),
    )(z1p, y1, degp, b1.reshape(1, H), W2p)

    z2p = _scat(y2, sd, zeros128)

    b2p = jnp.pad(b2, (0, CP - C)).reshape(1, CP)
    out = pl.pallas_call(
        _tc3_body,
        grid=(N // _BN,),
        in_specs=[pl.BlockSpec((NCORES, _BN, CP), lambda i: (0, i, 0)),
                  pl.BlockSpec((_BN, CP), lambda i: (i, 0)),
                  _deg_spec(),
                  pl.BlockSpec((1, CP), lambda i: (0, 0))],
        out_specs=pl.BlockSpec((_BN, CP), lambda i: (i, 0)),
        out_shape=jax.ShapeDtypeStruct((N, CP), jnp.float32),
    )(z2p, y2, degp, b2p)

    return out[:, :C]


# speedup vs baseline: 2.1896x; 1.4393x over previous
import functools

import jax
import jax.numpy as jnp
from jax import lax
from jax.experimental import pallas as pl
from jax.experimental.pallas import tpu as pltpu
from jax.experimental.pallas import tpu_sc as plsc

N = 10000
E = 320000
D = 128
H = 128
C = 40
CP = 64

NCORES = 2
NSUB = 16
NW = NCORES * NSUB
CHUNK = 128
NBUF = 2
TOTC = 160
CPW = TOTC // NCORES
HC = CPW // 2
NCH = NSUB * TOTC
EP = NCH * CHUNK
NACC = 10112
RPT = NACC // NSUB
TRASH = N

_mesh = plsc.VectorSubcoreMesh(core_axis_name="c", subcore_axis_name="s")


def _make_scatter(F, gather, n0=CPW, n1=CPW):
    @functools.partial(
        pl.kernel,
        out_type=jax.ShapeDtypeStruct((NCORES, NACC, F), jnp.float32),
        mesh=_mesh,
        scratch_types=[
            pltpu.VMEM((HC, 2, CHUNK), jnp.int32),
            pltpu.VMEM((NBUF, CHUNK, F), jnp.float32),
            pltpu.VMEM_SHARED((NACC, F), jnp.float32),
            pltpu.SemaphoreType.DMA,
            pltpu.SemaphoreType.DMA,
        ],
    )
    def scat(table_hbm, sd_hbm, zeros_hbm, out_hbm,
             idx, rows, acc, semg0, semg1):
        c = lax.axis_index("c")
        s = lax.axis_index("s")
        r0 = s * RPT
        base = s * TOTC + c * n0
        nch = jnp.where(c == 0, n0, n1)
        semg = (semg0, semg1)

        def fetch_rows(j, b):
            pltpu.async_copy(table_hbm.at[idx.at[j, 0]], rows.at[b], semg[b])

        def wait_rows(b):
            pltpu.make_async_copy(table_hbm.at[idx.at[0, 0]], rows.at[b],
                                  semg[b]).wait()

        pltpu.sync_copy(zeros_hbm.at[pl.ds(r0, RPT)], acc.at[pl.ds(r0, RPT)])
        if not gather:
            pltpu.sync_copy(table_hbm.at[pl.ds(0, CHUNK)], rows.at[0])
        plsc.subcore_barrier()

        def run_half(h):
            pltpu.sync_copy(sd_hbm.at[pl.ds(base + h * HC, HC)], idx)
            if gather:
                fetch_rows(0, 0)
                fetch_rows(1, 1)

                def body(g, carry):
                    for b in range(2):
                        j = g * 2 + b
                        wait_rows(b)
                        pltpu.sync_copy(rows.at[b], acc.at[idx.at[j, 1]],
                                        add=True)
                        fetch_rows(jnp.minimum(j + 2, HC - 1), b)
                    return carry

                lax.fori_loop(0, HC // 2, body, 0)
                wait_rows(0)
                wait_rows(1)
            else:
                def dbody(j, carry):
                    pltpu.sync_copy(rows.at[0], acc.at[idx.at[j, 1]],
                                    add=True)
                    return carry

                lax.fori_loop(0, HC, dbody, 0)

        for h in range(max(n0, n1) // HC):
            if (h + 1) * HC <= min(n0, n1):
                run_half(h)
            else:
                @pl.when(h * HC < nch)
                def _():
                    run_half(h)
        plsc.subcore_barrier()
        pltpu.sync_copy(acc.at[pl.ds(r0, RPT)],
                        out_hbm.at[c, pl.ds(r0, RPT)])

    return scat


_scat = _make_scatter(H, gather=True)
_deg_kernel = _make_scatter(H, gather=False)

TPT = N // NSUB


def _make_scatter_spmem(F):
    @functools.partial(
        pl.kernel,
        out_type=jax.ShapeDtypeStruct((NCORES, NACC, F), jnp.float32),
        mesh=_mesh,
        compiler_params=pltpu.CompilerParams(use_tc_tiling_on_sc=False),
        scratch_types=[
            pltpu.VMEM((HC, 2, CHUNK), jnp.int32),
            pltpu.VMEM((NBUF, CHUNK, F), jnp.float32),
            pltpu.VMEM_SHARED((N, F), jnp.float32),
            pltpu.VMEM_SHARED((NACC, F), jnp.float32),
            pltpu.SemaphoreType.DMA,
            pltpu.SemaphoreType.DMA,
        ],
    )
    def scat(table_hbm, sd_hbm, zeros_hbm, out_hbm,
             idx, rows, tab, acc, semg0, semg1):
        c = lax.axis_index("c")
        s = lax.axis_index("s")
        r0 = s * RPT
        base = s * TOTC + c * CPW
        semg = (semg0, semg1)

        def fetch_rows(j, b):
            pltpu.async_copy(tab.at[idx.at[j, 0]], rows.at[b], semg[b])

        def wait_rows(b):
            pltpu.make_async_copy(tab.at[idx.at[0, 0]], rows.at[b],
                                  semg[b]).wait()

        pltpu.sync_copy(zeros_hbm.at[pl.ds(r0, RPT)], acc.at[pl.ds(r0, RPT)])
        pltpu.sync_copy(table_hbm.at[pl.ds(s * TPT, TPT)],
                        tab.at[pl.ds(s * TPT, TPT)])
        plsc.subcore_barrier()

        for h in range(CPW // HC):
            pltpu.sync_copy(sd_hbm.at[pl.ds(base + h * HC, HC)], idx)
            fetch_rows(0, 0)
            fetch_rows(1, 1)

            def body(g, carry):
                for b in range(2):
                    j = g * 2 + b
                    wait_rows(b)
                    pltpu.sync_copy(rows.at[b], acc.at[idx.at[j, 1]],
                                    add=True)
                    fetch_rows(jnp.minimum(j + 2, HC - 1), b)
                return carry

            lax.fori_loop(0, HC // 2, body, 0)
            wait_rows(0)
            wait_rows(1)
        plsc.subcore_barrier()
        pltpu.sync_copy(acc.at[pl.ds(r0, RPT)],
                        out_hbm.at[c, pl.ds(r0, RPT)])

    return scat


_scat2 = _make_scatter_spmem(CP)

_BN = 1000


def _dis_block(deg_ref):
    d = deg_ref[0, :, 0:1] + deg_ref[1, :, 0:1] + 1.0
    return lax.rsqrt(d)


def _tc1_body(x_ref, w_ref, deg_ref, y_ref):
    dis = _dis_block(deg_ref)
    y_ref[...] = jnp.dot(x_ref[...], w_ref[...],
                         preferred_element_type=jnp.float32) * dis


def _tc2_body(z_ref, y1_ref, deg_ref, b1_ref, w2_ref, y2_ref):
    dis = _dis_block(deg_ref)
    h = (z_ref[0] + z_ref[1] + y1_ref[...]) * dis + b1_ref[...]
    h = jnp.maximum(h, 0.0)
    y2_ref[...] = jnp.dot(h, w2_ref[...],
                          preferred_element_type=jnp.float32) * dis


def _tc3_body(z_ref, y2_ref, deg_ref, b2_ref, out_ref):
    dis = _dis_block(deg_ref)
    v = (z_ref[0] + z_ref[1] + y2_ref[...]) * dis + b2_ref[...]
    col = lax.broadcasted_iota(jnp.int32, v.shape, 1)
    valid = col < C
    m = jnp.max(jnp.where(valid, v, -1e30), axis=1, keepdims=True)
    e = jnp.where(valid, jnp.exp(v - m), 0.0)
    lse = jnp.log(jnp.sum(e, axis=1, keepdims=True)) + m
    out_ref[...] = v - lse


def _deg_spec():
    return pl.BlockSpec((NCORES, _BN, H), lambda i: (0, i, 0))


def kernel(x, edge_index, W1, b1, W2, b2):
    src = edge_index[0].astype(jnp.int32)
    dst = edge_index[1].astype(jnp.int32)
    pad = EP - E
    srcw = jnp.concatenate([src, jnp.zeros((pad,), jnp.int32)]
                           ).reshape(NCH, CHUNK)
    dstw = jnp.concatenate([dst, jnp.full((pad,), TRASH, jnp.int32)]
                           ).reshape(NCH, CHUNK)
    sd = jnp.stack([srcw, dstw], axis=1)

    zeros128 = jnp.zeros((NACC, H), jnp.float32)
    ones128 = jnp.ones((CHUNK, H), jnp.float32)
    degp = _deg_kernel(ones128, sd, zeros128)

    y1 = pl.pallas_call(
        _tc1_body,
        grid=(N // _BN,),
        in_specs=[pl.BlockSpec((_BN, D), lambda i: (i, 0)),
                  pl.BlockSpec((D, H), lambda i: (0, 0)),
                  _deg_spec()],
        out_specs=pl.BlockSpec((_BN, H), lambda i: (i, 0)),
        out_shape=jax.ShapeDtypeStruct((N, H), jnp.float32),
    )(x, W1, degp)

    z1p = _scat(y1, sd, zeros128)

    W2p = jnp.pad(W2, ((0, 0), (0, CP - C)))
    y2 = pl.pallas_call(
        _tc2_body,
        grid=(N // _BN,),
        in_specs=[pl.BlockSpec((NCORES, _BN, H), lambda i: (0, i, 0)),
                  pl.BlockSpec((_BN, H), lambda i: (i, 0)),
                  _deg_spec(),
                  pl.BlockSpec((1, H), lambda i: (0, 0)),
                  pl.BlockSpec((H, CP), lambda i: (0, 0))],
        out_specs=pl.BlockSpec((_BN, CP), lambda i: (i, 0)),
        out_shape=jax.ShapeDtypeStruct((N, CP), jnp.float32),
    )(z1p, y1, degp, b1.reshape(1, H), W2p)

    zeros64 = jnp.zeros((NACC, CP), jnp.float32)
    z2p = _scat2(y2, sd, zeros64)

    b2p = jnp.pad(b2, (0, CP - C)).reshape(1, CP)
    out = pl.pallas_call(
        _tc3_body,
        grid=(N // _BN,),
        in_specs=[pl.BlockSpec((NCORES, _BN, CP), lambda i: (0, i, 0)),
                  pl.BlockSpec((_BN, CP), lambda i: (i, 0)),
                  _deg_spec(),
                  pl.BlockSpec((1, CP), lambda i: (0, 0))],
        out_specs=pl.BlockSpec((_BN, CP), lambda i: (i, 0)),
        out_shape=jax.ShapeDtypeStruct((N, CP), jnp.float32),
    )(z2p, y2, degp, b2p)

    return out[:, :C]


# speedup vs baseline: 3.1938x; 1.4586x over previous
import functools

import jax
import jax.numpy as jnp
from jax import lax
from jax.experimental import pallas as pl
from jax.experimental.pallas import tpu as pltpu
from jax.experimental.pallas import tpu_sc as plsc

N = 10000
E = 320000
D = 128
H = 128
C = 40
CP = 64

NCORES = 2
NSUB = 16
NW = NCORES * NSUB
CHUNK = 128
NBUF = 2
TOTC = 160
CPW = TOTC // NCORES
HC = CPW // 2
NCH = NSUB * TOTC
EP = NCH * CHUNK
NACC = 10112
RPT = NACC // NSUB
TRASH = N

_mesh = plsc.VectorSubcoreMesh(core_axis_name="c", subcore_axis_name="s")


def _make_scatter(F, gather, n0=CPW, n1=CPW):
    @functools.partial(
        pl.kernel,
        out_type=jax.ShapeDtypeStruct((NCORES, NACC, F), jnp.float32),
        mesh=_mesh,
        scratch_types=[
            pltpu.VMEM((HC, 2, CHUNK), jnp.int32),
            pltpu.VMEM((NBUF, CHUNK, F), jnp.float32),
            pltpu.VMEM_SHARED((NACC, F), jnp.float32),
            pltpu.SemaphoreType.DMA,
            pltpu.SemaphoreType.DMA,
        ],
    )
    def scat(table_hbm, sd_hbm, zeros_hbm, out_hbm,
             idx, rows, acc, semg0, semg1):
        c = lax.axis_index("c")
        s = lax.axis_index("s")
        r0 = s * RPT
        base = s * TOTC + c * n0
        nch = jnp.where(c == 0, n0, n1)
        semg = (semg0, semg1)

        def fetch_rows(j, b):
            pltpu.async_copy(table_hbm.at[idx.at[j, 0]], rows.at[b], semg[b])

        def wait_rows(b):
            pltpu.make_async_copy(table_hbm.at[idx.at[0, 0]], rows.at[b],
                                  semg[b]).wait()

        pltpu.sync_copy(zeros_hbm.at[pl.ds(r0, RPT)], acc.at[pl.ds(r0, RPT)])
        if not gather:
            pltpu.sync_copy(table_hbm.at[pl.ds(0, CHUNK)], rows.at[0])
        plsc.subcore_barrier()

        def run_half(h):
            pltpu.sync_copy(sd_hbm.at[pl.ds(base + h * HC, HC)], idx)
            if gather:
                fetch_rows(0, 0)
                fetch_rows(1, 1)

                def body(g, carry):
                    for b in range(2):
                        j = g * 2 + b
                        wait_rows(b)
                        pltpu.sync_copy(rows.at[b], acc.at[idx.at[j, 1]],
                                        add=True)
                        fetch_rows(jnp.minimum(j + 2, HC - 1), b)
                    return carry

                lax.fori_loop(0, HC // 2, body, 0)
                wait_rows(0)
                wait_rows(1)
            else:
                def dbody(j, carry):
                    pltpu.sync_copy(rows.at[0], acc.at[idx.at[j, 1]],
                                    add=True)
                    return carry

                lax.fori_loop(0, HC, dbody, 0)

        for h in range(max(n0, n1) // HC):
            if (h + 1) * HC <= min(n0, n1):
                run_half(h)
            else:
                @pl.when(h * HC < nch)
                def _():
                    run_half(h)
        plsc.subcore_barrier()
        pltpu.sync_copy(acc.at[pl.ds(r0, RPT)],
                        out_hbm.at[c, pl.ds(r0, RPT)])

    return scat


_scat = _make_scatter(H, gather=True)
_deg_kernel = _make_scatter(H, gather=False)

TPT = N // NSUB


def _make_scatter_spmem(F):
    @functools.partial(
        pl.kernel,
        out_type=jax.ShapeDtypeStruct((NCORES, NACC, F), jnp.float32),
        mesh=_mesh,
        compiler_params=pltpu.CompilerParams(use_tc_tiling_on_sc=False),
        scratch_types=[
            pltpu.VMEM((HC, 2, CHUNK), jnp.int32),
            pltpu.VMEM((NBUF, CHUNK, F), jnp.float32),
            pltpu.VMEM_SHARED((N, F), jnp.float32),
            pltpu.VMEM_SHARED((NACC, F), jnp.float32),
            pltpu.SemaphoreType.DMA,
            pltpu.SemaphoreType.DMA,
        ],
    )
    def scat(table_hbm, sd_hbm, zeros_hbm, out_hbm,
             idx, rows, tab, acc, semg0, semg1):
        c = lax.axis_index("c")
        s = lax.axis_index("s")
        r0 = s * RPT
        base = s * TOTC + c * CPW
        semg = (semg0, semg1)

        def fetch_rows(j, b):
            pltpu.async_copy(tab.at[idx.at[j, 0]], rows.at[b], semg[b])

        def wait_rows(b):
            pltpu.make_async_copy(tab.at[idx.at[0, 0]], rows.at[b],
                                  semg[b]).wait()

        pltpu.sync_copy(zeros_hbm.at[pl.ds(r0, RPT)], acc.at[pl.ds(r0, RPT)])
        pltpu.sync_copy(table_hbm.at[pl.ds(s * TPT, TPT)],
                        tab.at[pl.ds(s * TPT, TPT)])
        plsc.subcore_barrier()

        for h in range(CPW // HC):
            pltpu.sync_copy(sd_hbm.at[pl.ds(base + h * HC, HC)], idx)
            fetch_rows(0, 0)
            fetch_rows(1, 1)

            def body(g, carry):
                for b in range(2):
                    j = g * 2 + b
                    wait_rows(b)
                    pltpu.sync_copy(rows.at[b], acc.at[idx.at[j, 1]],
                                    add=True)
                    fetch_rows(jnp.minimum(j + 2, HC - 1), b)
                return carry

            lax.fori_loop(0, HC // 2, body, 0)
            wait_rows(0)
            wait_rows(1)
        plsc.subcore_barrier()
        pltpu.sync_copy(acc.at[pl.ds(r0, RPT)],
                        out_hbm.at[c, pl.ds(r0, RPT)])

    return scat


_scat2 = _make_scatter_spmem(CP)

_BN = 1000


def _dis_block(deg_ref):
    d = deg_ref[0, :, 0:1] + deg_ref[1, :, 0:1] + 1.0
    return lax.rsqrt(d)


def _tc1_body(x_ref, w_ref, deg_ref, y_ref):
    dis = _dis_block(deg_ref)
    y_ref[...] = jnp.dot(x_ref[...], w_ref[...],
                         preferred_element_type=jnp.float32) * dis


def _tc2_body(za_ref, zb_ref, y1_ref, deg_ref, b1_ref, w2_ref, y2_ref):
    dis = _dis_block(deg_ref)
    b1 = b1_ref[...]
    y1 = y1_ref[...]
    ha = jnp.maximum((za_ref[0] + za_ref[1] + y1[:, :CP]) * dis + b1[:, :CP],
                     0.0)
    hb = jnp.maximum((zb_ref[0] + zb_ref[1] + y1[:, CP:]) * dis + b1[:, CP:],
                     0.0)
    y2_ref[...] = (jnp.dot(ha, w2_ref[:CP, :],
                           preferred_element_type=jnp.float32) +
                   jnp.dot(hb, w2_ref[CP:, :],
                           preferred_element_type=jnp.float32)) * dis


def _tc3_body(z_ref, y2_ref, deg_ref, b2_ref, out_ref):
    dis = _dis_block(deg_ref)
    v = (z_ref[0] + z_ref[1] + y2_ref[...]) * dis + b2_ref[...]
    col = lax.broadcasted_iota(jnp.int32, v.shape, 1)
    valid = col < C
    m = jnp.max(jnp.where(valid, v, -1e30), axis=1, keepdims=True)
    e = jnp.where(valid, jnp.exp(v - m), 0.0)
    lse = jnp.log(jnp.sum(e, axis=1, keepdims=True)) + m
    out_ref[...] = v - lse


def _deg_spec():
    return pl.BlockSpec((NCORES, _BN, H), lambda i: (0, i, 0))


def kernel(x, edge_index, W1, b1, W2, b2):
    src = edge_index[0].astype(jnp.int32)
    dst = edge_index[1].astype(jnp.int32)
    pad = EP - E
    srcw = jnp.concatenate([src, jnp.zeros((pad,), jnp.int32)]
                           ).reshape(NCH, CHUNK)
    dstw = jnp.concatenate([dst, jnp.full((pad,), TRASH, jnp.int32)]
                           ).reshape(NCH, CHUNK)
    sd = jnp.stack([srcw, dstw], axis=1)

    zeros128 = jnp.zeros((NACC, H), jnp.float32)
    ones128 = jnp.ones((CHUNK, H), jnp.float32)
    degp = _deg_kernel(ones128, sd, zeros128)

    y1 = pl.pallas_call(
        _tc1_body,
        grid=(N // _BN,),
        in_specs=[pl.BlockSpec((_BN, D), lambda i: (i, 0)),
                  pl.BlockSpec((D, H), lambda i: (0, 0)),
                  _deg_spec()],
        out_specs=pl.BlockSpec((_BN, H), lambda i: (i, 0)),
        out_shape=jax.ShapeDtypeStruct((N, H), jnp.float32),
    )(x, W1, degp)

    zeros64 = jnp.zeros((NACC, CP), jnp.float32)
    z1pa = _scat2(y1[:, :CP], sd, zeros64)
    z1pb = _scat2(y1[:, CP:], sd, zeros64)

    W2p = jnp.pad(W2, ((0, 0), (0, CP - C)))
    y2 = pl.pallas_call(
        _tc2_body,
        grid=(N // _BN,),
        in_specs=[pl.BlockSpec((NCORES, _BN, CP), lambda i: (0, i, 0)),
                  pl.BlockSpec((NCORES, _BN, CP), lambda i: (0, i, 0)),
                  pl.BlockSpec((_BN, H), lambda i: (i, 0)),
                  _deg_spec(),
                  pl.BlockSpec((1, H), lambda i: (0, 0)),
                  pl.BlockSpec((H, CP), lambda i: (0, 0))],
        out_specs=pl.BlockSpec((_BN, CP), lambda i: (i, 0)),
        out_shape=jax.ShapeDtypeStruct((N, CP), jnp.float32),
    )(z1pa, z1pb, y1, degp, b1.reshape(1, H), W2p)

    z2p = _scat2(y2, sd, zeros64)

    b2p = jnp.pad(b2, (0, CP - C)).reshape(1, CP)
    out = pl.pallas_call(
        _tc3_body,
        grid=(N // _BN,),
        in_specs=[pl.BlockSpec((NCORES, _BN, CP), lambda i: (0, i, 0)),
                  pl.BlockSpec((_BN, CP), lambda i: (i, 0)),
                  _deg_spec(),
                  pl.BlockSpec((1, CP), lambda i: (0, 0))],
        out_specs=pl.BlockSpec((_BN, CP), lambda i: (i, 0)),
        out_shape=jax.ShapeDtypeStruct((N, CP), jnp.float32),
    )(z2p, y2, degp, b2p)

    return out[:, :C]


# speedup vs baseline: 3.3482x; 1.0483x over previous
import functools

import jax
import jax.numpy as jnp
from jax import lax
from jax.experimental import pallas as pl
from jax.experimental.pallas import tpu as pltpu
from jax.experimental.pallas import tpu_sc as plsc

N = 10000
E = 320000
D = 128
H = 128
C = 40
CP = 64

NCORES = 2
NSUB = 16
NW = NCORES * NSUB
CHUNK = 128
NBUF = 2
TOTC = 160
CPW = TOTC // NCORES
HC = CPW // 2
NCH = NSUB * TOTC
EP = NCH * CHUNK
NACC = 10112
RPT = NACC // NSUB
TRASH = N

_mesh = plsc.VectorSubcoreMesh(core_axis_name="c", subcore_axis_name="s")


def _make_scatter(F, gather, n0=CPW, n1=CPW):
    @functools.partial(
        pl.kernel,
        out_type=jax.ShapeDtypeStruct((NCORES, NACC, F), jnp.float32),
        mesh=_mesh,
        scratch_types=[
            pltpu.VMEM((HC, 2, CHUNK), jnp.int32),
            pltpu.VMEM((NBUF, CHUNK, F), jnp.float32),
            pltpu.VMEM_SHARED((NACC, F), jnp.float32),
            pltpu.SemaphoreType.DMA,
            pltpu.SemaphoreType.DMA,
        ],
    )
    def scat(table_hbm, sd_hbm, zeros_hbm, out_hbm,
             idx, rows, acc, semg0, semg1):
        c = lax.axis_index("c")
        s = lax.axis_index("s")
        r0 = s * RPT
        base = s * TOTC + c * n0
        nch = jnp.where(c == 0, n0, n1)
        semg = (semg0, semg1)

        def fetch_rows(j, b):
            pltpu.async_copy(table_hbm.at[idx.at[j, 0]], rows.at[b], semg[b])

        def wait_rows(b):
            pltpu.make_async_copy(table_hbm.at[idx.at[0, 0]], rows.at[b],
                                  semg[b]).wait()

        pltpu.sync_copy(zeros_hbm.at[pl.ds(r0, RPT)], acc.at[pl.ds(r0, RPT)])
        if not gather:
            pltpu.sync_copy(table_hbm.at[pl.ds(0, CHUNK)], rows.at[0])
        plsc.subcore_barrier()

        def run_half(h):
            pltpu.sync_copy(sd_hbm.at[pl.ds(base + h * HC, HC)], idx)
            if gather:
                fetch_rows(0, 0)
                fetch_rows(1, 1)

                def body(g, carry):
                    for b in range(2):
                        j = g * 2 + b
                        wait_rows(b)
                        pltpu.sync_copy(rows.at[b], acc.at[idx.at[j, 1]],
                                        add=True)
                        fetch_rows(jnp.minimum(j + 2, HC - 1), b)
                    return carry

                lax.fori_loop(0, HC // 2, body, 0)
                wait_rows(0)
                wait_rows(1)
            else:
                def dbody(j, carry):
                    pltpu.sync_copy(rows.at[0], acc.at[idx.at[j, 1]],
                                    add=True)
                    return carry

                lax.fori_loop(0, HC, dbody, 0)

        for h in range(max(n0, n1) // HC):
            if (h + 1) * HC <= min(n0, n1):
                run_half(h)
            else:
                @pl.when(h * HC < nch)
                def _():
                    run_half(h)
        plsc.subcore_barrier()
        pltpu.sync_copy(acc.at[pl.ds(r0, RPT)],
                        out_hbm.at[c, pl.ds(r0, RPT)])

    return scat


_scat = _make_scatter(H, gather=True)
_deg_kernel = _make_scatter(H, gather=False)

TPT = N // NSUB


def _make_scatter_spmem(F, gather=True):
    @functools.partial(
        pl.kernel,
        out_type=jax.ShapeDtypeStruct((NCORES, NACC, F), jnp.float32),
        mesh=_mesh,
        compiler_params=pltpu.CompilerParams(use_tc_tiling_on_sc=False),
        scratch_types=[
            pltpu.VMEM((HC, 2, CHUNK), jnp.int32),
            pltpu.VMEM((NBUF, CHUNK, F), jnp.float32),
            pltpu.VMEM_SHARED((N, F), jnp.float32),
            pltpu.VMEM_SHARED((NACC, F), jnp.float32),
            pltpu.SemaphoreType.DMA,
            pltpu.SemaphoreType.DMA,
        ],
    )
    def scat(table_hbm, sd_hbm, zeros_hbm, out_hbm,
             idx, rows, tab, acc, semg0, semg1):
        c = lax.axis_index("c")
        s = lax.axis_index("s")
        r0 = s * RPT
        base = s * TOTC + c * CPW
        semg = (semg0, semg1)

        def fetch_rows(j, b):
            pltpu.async_copy(tab.at[idx.at[j, 0]], rows.at[b], semg[b])

        def wait_rows(b):
            pltpu.make_async_copy(tab.at[idx.at[0, 0]], rows.at[b],
                                  semg[b]).wait()

        pltpu.sync_copy(zeros_hbm.at[pl.ds(r0, RPT)], acc.at[pl.ds(r0, RPT)])
        if gather:
            pltpu.sync_copy(table_hbm.at[pl.ds(s * TPT, TPT)],
                            tab.at[pl.ds(s * TPT, TPT)])
        else:
            pltpu.sync_copy(table_hbm.at[pl.ds(0, CHUNK)], rows.at[0])
        plsc.subcore_barrier()

        for h in range(CPW // HC):
            pltpu.sync_copy(sd_hbm.at[pl.ds(base + h * HC, HC)], idx)
            if gather:
                fetch_rows(0, 0)
                fetch_rows(1, 1)

                def body(g, carry):
                    for b in range(2):
                        j = g * 2 + b
                        wait_rows(b)
                        pltpu.sync_copy(rows.at[b], acc.at[idx.at[j, 1]],
                                        add=True)
                        fetch_rows(jnp.minimum(j + 2, HC - 1), b)
                    return carry

                lax.fori_loop(0, HC // 2, body, 0)
                wait_rows(0)
                wait_rows(1)
            else:
                def dbody(j, carry):
                    pltpu.sync_copy(rows.at[0], acc.at[idx.at[j, 1]],
                                    add=True)
                    return carry

                lax.fori_loop(0, HC, dbody, 0)
        plsc.subcore_barrier()
        pltpu.sync_copy(acc.at[pl.ds(r0, RPT)],
                        out_hbm.at[c, pl.ds(r0, RPT)])

    return scat


_scat2 = _make_scatter_spmem(CP)
_deg64 = _make_scatter_spmem(CP, gather=False)

_BN = 1000


def _dis_block(deg_ref):
    d = deg_ref[0, :, 0:1] + deg_ref[1, :, 0:1] + 1.0
    return lax.rsqrt(d)


def _tc1_body(x_ref, w_ref, deg_ref, y_ref):
    dis = _dis_block(deg_ref)
    y_ref[...] = jnp.dot(x_ref[...], w_ref[...],
                         preferred_element_type=jnp.float32) * dis


def _tc2_body(za_ref, zb_ref, y1_ref, deg_ref, b1_ref, w2_ref, y2_ref):
    dis = _dis_block(deg_ref)
    b1 = b1_ref[...]
    y1 = y1_ref[...]
    ha = jnp.maximum((za_ref[0] + za_ref[1] + y1[:, :CP]) * dis + b1[:, :CP],
                     0.0)
    hb = jnp.maximum((zb_ref[0] + zb_ref[1] + y1[:, CP:]) * dis + b1[:, CP:],
                     0.0)
    y2_ref[...] = (jnp.dot(ha, w2_ref[:CP, :],
                           preferred_element_type=jnp.float32) +
                   jnp.dot(hb, w2_ref[CP:, :],
                           preferred_element_type=jnp.float32)) * dis


def _tc3_body(z_ref, y2_ref, deg_ref, b2_ref, out_ref):
    dis = _dis_block(deg_ref)
    v = (z_ref[0] + z_ref[1] + y2_ref[...]) * dis + b2_ref[...]
    col = lax.broadcasted_iota(jnp.int32, v.shape, 1)
    valid = col < C
    m = jnp.max(jnp.where(valid, v, -1e30), axis=1, keepdims=True)
    e = jnp.where(valid, jnp.exp(v - m), 0.0)
    lse = jnp.log(jnp.sum(e, axis=1, keepdims=True)) + m
    out_ref[...] = v - lse


def _deg_spec():
    return pl.BlockSpec((NCORES, _BN, CP), lambda i: (0, i, 0))


def kernel(x, edge_index, W1, b1, W2, b2):
    src = edge_index[0].astype(jnp.int32)
    dst = edge_index[1].astype(jnp.int32)
    pad = EP - E
    srcw = jnp.concatenate([src, jnp.zeros((pad,), jnp.int32)]
                           ).reshape(NCH, CHUNK)
    dstw = jnp.concatenate([dst, jnp.full((pad,), TRASH, jnp.int32)]
                           ).reshape(NCH, CHUNK)
    sd = jnp.stack([srcw, dstw], axis=1)

    zeros64 = jnp.zeros((NACC, CP), jnp.float32)
    ones64 = jnp.ones((CHUNK, CP), jnp.float32)
    degp = _deg64(ones64, sd, zeros64)

    y1 = pl.pallas_call(
        _tc1_body,
        grid=(N // _BN,),
        in_specs=[pl.BlockSpec((_BN, D), lambda i: (i, 0)),
                  pl.BlockSpec((D, H), lambda i: (0, 0)),
                  _deg_spec()],
        out_specs=pl.BlockSpec((_BN, H), lambda i: (i, 0)),
        out_shape=jax.ShapeDtypeStruct((N, H), jnp.float32),
    )(x, W1, degp)

    z1pa = _scat2(y1[:, :CP], sd, zeros64)
    z1pb = _scat2(y1[:, CP:], sd, zeros64)

    W2p = jnp.pad(W2, ((0, 0), (0, CP - C)))
    y2 = pl.pallas_call(
        _tc2_body,
        grid=(N // _BN,),
        in_specs=[pl.BlockSpec((NCORES, _BN, CP), lambda i: (0, i, 0)),
                  pl.BlockSpec((NCORES, _BN, CP), lambda i: (0, i, 0)),
                  pl.BlockSpec((_BN, H), lambda i: (i, 0)),
                  _deg_spec(),
                  pl.BlockSpec((1, H), lambda i: (0, 0)),
                  pl.BlockSpec((H, CP), lambda i: (0, 0))],
        out_specs=pl.BlockSpec((_BN, CP), lambda i: (i, 0)),
        out_shape=jax.ShapeDtypeStruct((N, CP), jnp.float32),
    )(z1pa, z1pb, y1, degp, b1.reshape(1, H), W2p)

    z2p = _scat2(y2, sd, zeros64)

    b2p = jnp.pad(b2, (0, CP - C)).reshape(1, CP)
    out = pl.pallas_call(
        _tc3_body,
        grid=(N // _BN,),
        in_specs=[pl.BlockSpec((NCORES, _BN, CP), lambda i: (0, i, 0)),
                  pl.BlockSpec((_BN, CP), lambda i: (i, 0)),
                  _deg_spec(),
                  pl.BlockSpec((1, CP), lambda i: (0, 0))],
        out_specs=pl.BlockSpec((_BN, CP), lambda i: (i, 0)),
        out_shape=jax.ShapeDtypeStruct((N, CP), jnp.float32),
    )(z2p, y2, degp, b2p)

    return out[:, :C]
